# baseline (speedup 1.0000x reference)
import jax
import jax.numpy as jnp
from jax.experimental import pallas as pl
from jax.experimental.pallas import tpu as pltpu
from jax.experimental.pallas import tpu_sc as plsc

BATCH = 16384
FACTOR = 32
MLP_DIM = 128
GATHER_WINDOW = 256

def _vector_mesh():
    return plsc.VectorSubcoreMesh(
        core_axis_name="core", subcore_axis_name="subcore"
    )


def _gather_pipeline(table_hbm, idx_hbm, out_hbm):
    def body(idx_vmem, out_vmem):
        pltpu.sync_copy(table_hbm.at[idx_vmem.at[0]], out_vmem)

    pltpu.emit_pipeline(
        body,
        grid=(BATCH // GATHER_WINDOW,),
        in_specs=[pl.BlockSpec((1, GATHER_WINDOW), index_map=lambda i: (0, i))],
        out_specs=[pl.BlockSpec((GATHER_WINDOW, 128), index_map=lambda i: (i, 0))],
        core_axis_name=("core", "subcore"),
        dimension_semantics=(pltpu.PARALLEL,),
    )(idx_hbm, out_hbm)


def _sc_gather_mlp(user2, item2, W_user_mlp, W_item_mlp):
    out_types = (
        jax.ShapeDtypeStruct((BATCH, MLP_DIM), jnp.float32),
        jax.ShapeDtypeStruct((BATCH, MLP_DIM), jnp.float32),
    )

    @pl.kernel(out_type=out_types, mesh=_vector_mesh(), scratch_types=[])
    def gather_mlp(u_hbm, i_hbm, wum_hbm, wim_hbm, eum_hbm, eim_hbm):
        _gather_pipeline(wum_hbm, u_hbm, eum_hbm)
        _gather_pipeline(wim_hbm, i_hbm, eim_hbm)

    return gather_mlp(user2, item2, W_user_mlp, W_item_mlp)


def _sc_gather_gmf(u2half, i2half, Wpack):
    out_types = (
        jax.ShapeDtypeStruct((BATCH, 128), jnp.float32),
        jax.ShapeDtypeStruct((BATCH, 128), jnp.float32),
    )

    @pl.kernel(out_type=out_types, mesh=_vector_mesh(), scratch_types=[])
    def gather_gmf(u_hbm, i_hbm, wp_hbm, gu_hbm, gi_hbm):
        _gather_pipeline(wp_hbm, u_hbm, gu_hbm)
        _gather_pipeline(wp_hbm, i_hbm, gi_hbm)

    return gather_gmf(u2half, i2half, Wpack)


def _tc_dense_kernel(gu_ref, gi_ref, par_ref, eum_ref, eim_ref,
                     w0a_ref, w0b_ref, b0_ref, w1_ref, b1_ref,
                     w2_ref, b2_ref, pwg_ref, pwm_ref, pb_ref, out_ref):
    h0 = jnp.dot(eum_ref[...], w0a_ref[...], preferred_element_type=jnp.float32)
    h0 += jnp.dot(eim_ref[...], w0b_ref[...], preferred_element_type=jnp.float32)
    h0 = jnp.maximum(h0 + b0_ref[...], 0.0)
    h1 = jnp.dot(h0, w1_ref[...], preferred_element_type=jnp.float32)
    h1 = jnp.maximum(h1 + b1_ref[...], 0.0)
    h2 = jnp.dot(h1, w2_ref[...], preferred_element_type=jnp.float32)
    h2 = jnp.maximum(h2 + b2_ref[...], 0.0)

    gu = gu_ref[...]
    gi = gi_ref[...]
    ue = par_ref[:, 0:1]
    uo = par_ref[:, 1:2]
    ie = par_ref[:, 2:3]
    io = par_ref[:, 3:4]
    gu_al = gu * ue + jnp.roll(gu * uo, -64, axis=1)
    gi_al = jnp.roll(gi * ie, -32, axis=1) + jnp.roll(gi * io, -96, axis=1)
    g = gu_al * gi_al

    pred = jnp.dot(g, pwg_ref[...], preferred_element_type=jnp.float32)
    pred += jnp.dot(h2, pwm_ref[...], preferred_element_type=jnp.float32)
    out_ref[...] = pred + pb_ref[0, 0]


def kernel(user, item, W_user_gmf, W_item_gmf, W_user_mlp, W_item_mlp,
           mlp_W0, mlp_b0, mlp_W1, mlp_b1, mlp_W2, mlp_b2, pred_W, pred_b):
    user = user.astype(jnp.int32)
    item = item.astype(jnp.int32)
    user2 = user.reshape(1, BATCH)
    item2 = item.reshape(1, BATCH)

    Wpack = jnp.concatenate([W_user_gmf, W_item_gmf], axis=1).reshape(-1, 128)
    u2half = (user2 // 2)
    i2half = (item2 // 2)
    par = jnp.stack(
        [1.0 - (user % 2), (user % 2).astype(jnp.float32),
         1.0 - (item % 2), (item % 2).astype(jnp.float32)], axis=1
    ).astype(jnp.float32)

    eu_mlp, ei_mlp = _sc_gather_mlp(user2, item2, W_user_mlp, W_item_mlp)
    gu, gi = _sc_gather_gmf(u2half, i2half, Wpack)

    w0a = mlp_W0[:, :MLP_DIM].T
    w0b = mlp_W0[:, MLP_DIM:].T
    w1 = mlp_W1.T
    w2 = mlp_W2.T
    b0 = mlp_b0.reshape(1, -1)
    b1 = mlp_b1.reshape(1, -1)
    b2 = mlp_b2.reshape(1, -1)
    pwg = jnp.pad(pred_W[:, :FACTOR], ((0, 0), (0, 128 - FACTOR))).T
    pwm = pred_W[:, FACTOR:].T
    pb = pred_b.reshape(1, 1)

    blk = 2048
    grid = (BATCH // blk,)
    row_spec = lambda d: pl.BlockSpec((blk, d), lambda i: (i, 0))
    full = lambda a: pl.BlockSpec(a.shape, lambda i: (0,) * a.ndim)

    out = pl.pallas_call(
        _tc_dense_kernel,
        grid=grid,
        in_specs=[
            row_spec(128), row_spec(128), row_spec(4),
            row_spec(MLP_DIM), row_spec(MLP_DIM),
            full(w0a), full(w0b), full(b0),
            full(w1), full(b1), full(w2), full(b2),
            full(pwg), full(pwm), full(pb),
        ],
        out_specs=pl.BlockSpec((blk, 1), lambda i: (i, 0)),
        out_shape=jax.ShapeDtypeStruct((BATCH, 1), jnp.float32),
    )(gu, gi, par, eu_mlp, ei_mlp,
      w0a, w0b, b0, w1, b1, w2, b2, pwg, pwm, pb)
    return out.reshape(-1)
